# Initial kernel scaffold; baseline (speedup 1.0000x reference)
import jax
import jax.numpy as jnp
from jax.experimental import pallas as pl


def kernel(vocab_ids, labels, edge_lists, emb_table, W_self, W_nbr, b_gnn, W_out, b_out):
    raise NotImplementedError("write your pallas kernel here")


# speedup vs baseline: 3.0134x; 3.0134x over previous
import functools

import jax
import jax.numpy as jnp
from jax import lax
from jax.experimental import pallas as pl
from jax.experimental.pallas import tpu as pltpu
from jax.experimental.pallas import tpu_sc as plsc

N = 10000
D = 128
E = 320000
NW = 32
N_PAD = 10240
E_PAD = 327680
ROWS_W = N_PAD // NW
EDGES_W = E_PAD // NW
CHUNK = 128
N_CHUNKS = EDGES_W // CHUNK
STRIPE = N_PAD // 16


def _sc_mesh():
    return plsc.VectorSubcoreMesh(core_axis_name="c", subcore_axis_name="s")


def _emb_gather(vid_pad, emb_table):
    @functools.partial(
        pl.kernel,
        out_type=jax.ShapeDtypeStruct((N_PAD, D), jnp.float32),
        mesh=_sc_mesh(),
        scratch_types=[
            pltpu.VMEM((ROWS_W,), jnp.int32),
            pltpu.VMEM((ROWS_W, D), jnp.float32),
            pltpu.SemaphoreType.DMA,
        ],
    )
    def k(vid_hbm, emb_hbm, out_hbm, idx_v, rows_v, sem):
        wid = lax.axis_index("s") * 2 + lax.axis_index("c")
        base = wid * ROWS_W
        pltpu.sync_copy(vid_hbm.at[pl.ds(base, ROWS_W)], idx_v)
        pltpu.async_copy(emb_hbm.at[idx_v], rows_v, sem).wait()
        pltpu.sync_copy(rows_v, out_hbm.at[pl.ds(base, ROWS_W)])

    return k(vid_pad, emb_table)


def _edge_prop(raw_pad, src_pad, dst_pad, zblk):
    @functools.partial(
        pl.kernel,
        out_type=jax.ShapeDtypeStruct((2, N_PAD, D), jnp.float32),
        mesh=_sc_mesh(),
        scratch_types=[
            pltpu.VMEM((CHUNK,), jnp.int32),
            pltpu.VMEM((CHUNK,), jnp.int32),
            pltpu.VMEM((CHUNK, D), jnp.float32),
            pltpu.VMEM_SHARED((N_PAD, D), jnp.float32),
            pltpu.SemaphoreType.DMA,
        ],
    )
    def k(raw_hbm, src_hbm, dst_hbm, z_hbm, out_hbm, sidx_v, didx_v, rows_v,
          acc_sh, sem):
        cid = lax.axis_index("c")
        sid = lax.axis_index("s")
        wid = sid * 2 + cid
        base = wid * EDGES_W

        pltpu.sync_copy(z_hbm, acc_sh.at[pl.ds(sid * STRIPE, STRIPE)])
        plsc.subcore_barrier()

        def body(g, carry):
            off = base + g * CHUNK
            pltpu.sync_copy(src_hbm.at[pl.ds(off, CHUNK)], sidx_v)
            pltpu.sync_copy(dst_hbm.at[pl.ds(off, CHUNK)], didx_v)
            pltpu.async_copy(raw_hbm.at[sidx_v], rows_v, sem).wait()
            pltpu.sync_copy(rows_v, acc_sh.at[didx_v], add=True)
            return carry

        lax.fori_loop(0, N_CHUNKS, body, 0)

        plsc.subcore_barrier()
        pltpu.sync_copy(acc_sh.at[pl.ds(sid * STRIPE, STRIPE)],
                        out_hbm.at[cid, pl.ds(sid * STRIPE, STRIPE)])

    return k(raw_pad, src_pad, dst_pad, zblk)


def _tc_head(raw_in, partials, labels2, W_self, W_nbr, b_gnn2, W_out, b_out2):
    def body(raw_ref, p_ref, lab_ref, ws_ref, wn_ref, bg_ref, wo_ref, bo_ref,
             logits_ref, loss_ref):
        raw = raw_ref[...]
        agg = p_ref[0] + p_ref[1]
        x = (jnp.dot(raw, ws_ref[...], preferred_element_type=jnp.float32)
             + jnp.dot(agg, wn_ref[...], preferred_element_type=jnp.float32)
             + bg_ref[...])
        x = jnp.maximum(x, 0.0)
        wo = wo_ref[...]
        logits = (jnp.dot(raw, wo[:D], preferred_element_type=jnp.float32)
                  + jnp.dot(x, wo[D:], preferred_element_type=jnp.float32)
                  + bo_ref[...])
        logits_ref[...] = logits
        m = jnp.max(logits, axis=-1, keepdims=True)
        lse = jnp.log(jnp.sum(jnp.exp(logits - m), axis=-1, keepdims=True)) + m
        cls = lax.broadcasted_iota(jnp.int32, logits.shape, 1)
        picked = jnp.sum(jnp.where(cls == lab_ref[...], logits, 0.0),
                         axis=-1, keepdims=True)
        loss_ref[...] = jnp.sum(lse - picked, axis=0, keepdims=True) / N

    return pl.pallas_call(
        body,
        out_shape=(
            jax.ShapeDtypeStruct((N, 10), jnp.float32),
            jax.ShapeDtypeStruct((1, 1), jnp.float32),
        ),
    )(raw_in, partials, labels2, W_self, W_nbr, b_gnn2, W_out, b_out2)


def kernel(vocab_ids, labels, edge_lists, emb_table, W_self, W_nbr, b_gnn,
           W_out, b_out):
    vid = vocab_ids.astype(jnp.int32)
    vid_pad = jnp.pad(vid, (0, N_PAD - N))
    raw_pad = _emb_gather(vid_pad, emb_table)

    src = edge_lists[0].astype(jnp.int32)
    dst = edge_lists[1].astype(jnp.int32)
    src_pad = jnp.pad(src, (0, E_PAD - E))
    dst_pad = jnp.pad(dst, (0, E_PAD - E), constant_values=N_PAD - 1)
    zblk = jnp.zeros((STRIPE, D), jnp.float32)

    partials = _edge_prop(raw_pad, src_pad, dst_pad, zblk)

    logits, loss2 = _tc_head(
        raw_pad[:N],
        partials[:, :N, :],
        labels.astype(jnp.int32).reshape(N, 1),
        W_self, W_nbr,
        b_gnn.reshape(1, D),
        W_out,
        b_out.reshape(1, 10),
    )
    return logits, loss2[0, 0]
